# baseline (speedup 1.0000x reference)
import functools

import jax
import jax.numpy as jnp
from jax import lax
from jax.experimental import pallas as pl
from jax.experimental.pallas import tpu as pltpu
from jax.experimental.pallas import tpu_sc as plsc


def _make_sc_onehot(B, M):
    nc, ns, lanes = 2, 16, 16
    assert B == nc * ns and M % lanes == 0

    @functools.partial(
        pl.kernel,
        mesh=plsc.VectorSubcoreMesh(core_axis_name="c", subcore_axis_name="s"),
        out_type=jax.ShapeDtypeStruct((B, M), jnp.int32),
        scratch_types=[
            pltpu.VMEM((B,), jnp.int32),
            pltpu.VMEM((M,), jnp.int32),
        ],
    )
    def _sc_onehot(t_hbm, tgt_hbm, tvec, row):
        wid = lax.axis_index("s") * nc + lax.axis_index("c")
        pltpu.sync_copy(t_hbm, tvec)
        lane = lax.broadcasted_iota(jnp.int32, (lanes,), 0)
        trow0 = tvec[pl.ds(0, lanes)]
        trow1 = tvec[pl.ds(lanes, lanes)]
        trow = jnp.where((wid // lanes) == 1, trow1, trow0)
        idxv = lane * 0 + (wid % lanes)
        tw_vec = lax.gather(
            trow, idxv[:, None],
            lax.GatherDimensionNumbers(
                offset_dims=(), collapsed_slice_dims=(0,),
                start_index_map=(0,)),
            slice_sizes=(1,),
            mode=lax.GatherScatterMode.PROMISE_IN_BOUNDS)
        for k in range(M // lanes):
            row[pl.ds(k * lanes, lanes)] = \
                jnp.where(k * lanes + lane == tw_vec, 1, 0)
        pltpu.sync_copy(row, tgt_hbm.at[wid])

    return _sc_onehot


def _dot_t(a, b):
    return lax.dot_general(a, b, (((1,), (1,)), ((), ())),
                           preferred_element_type=jnp.float32)


def _verif_block(f_ref, t_ref, bank0_ref, bank1_ref, wt_ref, b_ref,
                 out0_ref, out1_ref,
                 fw_scr, ffx_scr, a_scr,
                 *, block_m: int):
    j = pl.program_id(0)
    B = f_ref.shape[0]
    inv_b = 1.0 / B
    t = t_ref[...]

    @pl.when(j == 0)
    def _prologue():
        f = f_ref[...]
        wt = wt_ref[...]
        bias = b_ref[...]
        fb = f * inv_b
        fw = jnp.concatenate([f * wt[0, :][None, :], f * wt[1, :][None, :]],
                             axis=0)
        fw_scr[...] = fw
        tc = t.reshape(B, 1)
        p = (tc == t).astype(jnp.float32)
        rows = lax.broadcasted_iota(jnp.int32, (B, B), 0)
        cols = lax.broadcasted_iota(jnp.int32, (B, B), 1)
        before = jnp.where(cols < rows, p, 0.0)
        rep = (jnp.sum(before, axis=1) == 0.0).astype(jnp.float32)
        dm = lax.dot_general(p, fb, (((1,), (0,)), ((), ())),
                             preferred_element_type=jnp.float32)
        ffd = _dot_t(fw, dm)
        dm2 = dm * dm
        rc2_0 = jnp.sum(dm2 * wt[0, :][None, :], axis=1)
        rc2_1 = jnp.sum(dm2 * wt[1, :][None, :], axis=1)
        ffx = jnp.concatenate(
            [ffd, rc2_0[None, :], rc2_1[None, :],
             jnp.zeros((6, B), jnp.float32)], axis=0)
        ffx_scr[...] = ffx * rep[None, :]
        f2 = f * f
        a0 = jnp.sum(f2 * wt[0, :][None, :], axis=1) + bias[0, 0]
        a1 = jnp.sum(f2 * wt[1, :][None, :], axis=1) + bias[0, 1]
        a_scr[...] = jnp.concatenate(
            [a0[None, :], a1[None, :], jnp.zeros((6, B), jnp.float32)], axis=0)

    wt = wt_ref[...]
    fw = fw_scr[...]
    ffx = ffx_scr[...]
    a = a_scr[...]
    two_inv_b = 2.0 * inv_b

    outs0 = []
    outs1 = []
    for s, bank_ref in ((0, bank0_ref), (1, bank1_ref)):
        bank_blk = bank_ref[...]
        bank2 = bank_blk * bank_blk

        cross_b = _dot_t(fw, bank_blk)
        s_b = _dot_t(wt, bank2)
        m_cols = ((2 * j + s) * block_m
                  + lax.broadcasted_iota(jnp.int32, (B, block_m), 1))
        o_blk = (t.reshape(B, 1) == m_cols)
        of = o_blk.astype(jnp.float32)
        cx = lax.dot_general(ffx, of, (((1,), (0,)), ((), ())),
                             preferred_element_type=jnp.float32)

        outs = []
        for c in (0, 1):
            cb_c = cross_b[c * B:(c + 1) * B, :]
            sx1_c = two_inv_b * jnp.sum(of * cb_c, axis=0)
            s_c = s_b[c, :] + cx[2 * B + c, :] + sx1_c
            outs.append(a[c, :][:, None]
                        - 2.0 * (cb_c + cx[c * B:(c + 1) * B, :])
                        + s_c[None, :])
        outs0.append(outs[0])
        outs1.append(outs[1])

    out0_ref[...] = jnp.concatenate(outs0, axis=1)
    out1_ref[...] = jnp.concatenate(outs1, axis=1)


def kernel(features, targets, bank, W, b):
    B, D = features.shape
    M, _ = bank.shape
    C = W.shape[1]
    block_m = 256
    nsteps = M // (2 * block_m)
    grid = (nsteps,)

    t2d = targets.reshape(1, B).astype(jnp.int32)
    wt = W.T
    b2d = b.reshape(1, C)

    tgt = _make_sc_onehot(B, M)(targets.astype(jnp.int32))

    out0, out1 = pl.pallas_call(
        functools.partial(_verif_block, block_m=block_m),
        grid=grid,
        in_specs=[
            pl.BlockSpec((B, D), lambda j: (0, 0)),
            pl.BlockSpec((1, B), lambda j: (0, 0)),
            pl.BlockSpec((block_m, D), lambda j: (2 * j, 0)),
            pl.BlockSpec((block_m, D), lambda j: (2 * j + 1, 0)),
            pl.BlockSpec((C, D), lambda j: (0, 0)),
            pl.BlockSpec((1, C), lambda j: (0, 0)),
        ],
        out_specs=[
            pl.BlockSpec((B, 2 * block_m), lambda j: (0, j)),
            pl.BlockSpec((B, 2 * block_m), lambda j: (0, j)),
        ],
        out_shape=[
            jax.ShapeDtypeStruct((B, M), jnp.float32),
            jax.ShapeDtypeStruct((B, M), jnp.float32),
        ],
        scratch_shapes=[
            pltpu.VMEM((2 * B, D), jnp.float32),
            pltpu.VMEM((2 * B + 8, B), jnp.float32),
            pltpu.VMEM((8, B), jnp.float32),
        ],
        compiler_params=pltpu.CompilerParams(
            dimension_semantics=("arbitrary",)),
    )(features, t2d, bank, bank, wt, b2d)

    bank_outputs = jnp.stack([out0, out1], axis=-1).reshape(B * M, C)
    bank_targets = tgt.reshape(-1)
    return bank_outputs, bank_targets


# speedup vs baseline: 2.2421x; 2.2421x over previous
import functools

import jax
import jax.numpy as jnp
from jax import lax
from jax.experimental import pallas as pl
from jax.experimental.pallas import tpu as pltpu


def _dot_t(a, b):
    return lax.dot_general(a, b, (((1,), (1,)), ((), ())),
                           preferred_element_type=jnp.float32)


def _verif_block(f_ref, t_ref, bank0_ref, bank1_ref, wt_ref, b_ref,
                 out0_ref, out1_ref, tgt_ref,
                 fw_scr, ffx_scr, a_scr,
                 *, block_m: int):
    j = pl.program_id(0)
    B = f_ref.shape[0]
    inv_b = 1.0 / B
    t = t_ref[...]

    @pl.when(j == 0)
    def _prologue():
        f = f_ref[...]
        wt = wt_ref[...]
        bias = b_ref[...]
        fb = f * inv_b
        fw = jnp.concatenate([f * wt[0, :][None, :], f * wt[1, :][None, :]],
                             axis=0)
        fw_scr[...] = fw
        tc = t.reshape(B, 1)
        p = (tc == t).astype(jnp.float32)
        rows = lax.broadcasted_iota(jnp.int32, (B, B), 0)
        cols = lax.broadcasted_iota(jnp.int32, (B, B), 1)
        before = jnp.where(cols < rows, p, 0.0)
        rep = (jnp.sum(before, axis=1) == 0.0).astype(jnp.float32)
        dm = lax.dot_general(p, fb, (((1,), (0,)), ((), ())),
                             preferred_element_type=jnp.float32)
        ffd = _dot_t(fw, dm)
        dm2 = dm * dm
        rc2_0 = jnp.sum(dm2 * wt[0, :][None, :], axis=1)
        rc2_1 = jnp.sum(dm2 * wt[1, :][None, :], axis=1)
        ffx = jnp.concatenate(
            [ffd, rc2_0[None, :], rc2_1[None, :],
             jnp.zeros((6, B), jnp.float32)], axis=0)
        ffx_scr[...] = ffx * rep[None, :]
        f2 = f * f
        a0 = jnp.sum(f2 * wt[0, :][None, :], axis=1) + bias[0, 0]
        a1 = jnp.sum(f2 * wt[1, :][None, :], axis=1) + bias[0, 1]
        a_scr[...] = jnp.concatenate(
            [a0[None, :], a1[None, :], jnp.zeros((6, B), jnp.float32)], axis=0)

    wt = wt_ref[...]
    fw = fw_scr[...]
    ffx = ffx_scr[...]
    a = a_scr[...]
    two_inv_b = 2.0 * inv_b

    outs0 = []
    outs1 = []
    tgts = []
    for s, bank_ref in ((0, bank0_ref), (1, bank1_ref)):
        bank_blk = bank_ref[...]
        bank2 = bank_blk * bank_blk

        cross_b = _dot_t(fw, bank_blk)
        s_b = _dot_t(wt, bank2)
        m_cols = ((2 * j + s) * block_m
                  + lax.broadcasted_iota(jnp.int32, (B, block_m), 1))
        o_blk = (t.reshape(B, 1) == m_cols)
        of = o_blk.astype(jnp.float32)
        cx = lax.dot_general(ffx, of, (((1,), (0,)), ((), ())),
                             preferred_element_type=jnp.float32)

        outs = []
        for c in (0, 1):
            cb_c = cross_b[c * B:(c + 1) * B, :]
            sx1_c = two_inv_b * jnp.sum(of * cb_c, axis=0)
            s_c = s_b[c, :] + cx[2 * B + c, :] + sx1_c
            outs.append(a[c, :][:, None]
                        - 2.0 * (cb_c + cx[c * B:(c + 1) * B, :])
                        + s_c[None, :])
        outs0.append(outs[0])
        outs1.append(outs[1])
        tgts.append(o_blk.astype(jnp.int32))

    out0_ref[...] = jnp.concatenate(outs0, axis=1)
    out1_ref[...] = jnp.concatenate(outs1, axis=1)
    tgt_ref[...] = jnp.concatenate(tgts, axis=1)


def kernel(features, targets, bank, W, b):
    B, D = features.shape
    M, _ = bank.shape
    C = W.shape[1]
    block_m = 256
    nsteps = M // (2 * block_m)
    grid = (nsteps,)

    t2d = targets.reshape(1, B).astype(jnp.int32)
    wt = W.T
    b2d = b.reshape(1, C)

    out0, out1, tgt = pl.pallas_call(
        functools.partial(_verif_block, block_m=block_m),
        grid=grid,
        in_specs=[
            pl.BlockSpec((B, D), lambda j: (0, 0)),
            pl.BlockSpec((1, B), lambda j: (0, 0)),
            pl.BlockSpec((block_m, D), lambda j: (2 * j, 0)),
            pl.BlockSpec((block_m, D), lambda j: (2 * j + 1, 0)),
            pl.BlockSpec((C, D), lambda j: (0, 0)),
            pl.BlockSpec((1, C), lambda j: (0, 0)),
        ],
        out_specs=[
            pl.BlockSpec((B, 2 * block_m), lambda j: (0, j)),
            pl.BlockSpec((B, 2 * block_m), lambda j: (0, j)),
            pl.BlockSpec((B, 2 * block_m), lambda j: (0, j)),
        ],
        out_shape=[
            jax.ShapeDtypeStruct((B, M), jnp.float32),
            jax.ShapeDtypeStruct((B, M), jnp.float32),
            jax.ShapeDtypeStruct((B, M), jnp.int32),
        ],
        scratch_shapes=[
            pltpu.VMEM((2 * B, D), jnp.float32),
            pltpu.VMEM((2 * B + 8, B), jnp.float32),
            pltpu.VMEM((8, B), jnp.float32),
        ],
        compiler_params=pltpu.CompilerParams(
            dimension_semantics=("arbitrary",)),
    )(features, t2d, bank, bank, wt, b2d)

    bank_outputs = jnp.stack([out0, out1], axis=-1).reshape(B * M, C)
    bank_targets = tgt.reshape(-1)
    return bank_outputs, bank_targets
